# baseline (speedup 1.0000x reference)
import functools

import jax
import jax.numpy as jnp
from jax import lax
from jax.experimental import pallas as pl
from jax.experimental.pallas import tpu as pltpu
from jax.experimental.pallas import tpu_sc as plsc

DIM = 64
NC, NS, L = 2, 16, 16
NW = NC * NS
BLK = 128


@functools.lru_cache(maxsize=None)
def _build_gather(B, H, vocab):
    n_blocks = (B // BLK) * H
    per_w = n_blocks // NW
    mesh = plsc.VectorSubcoreMesh(core_axis_name="c", subcore_axis_name="s")

    @functools.partial(
        pl.kernel,
        mesh=mesh,
        out_type=jax.ShapeDtypeStruct((H, 8, B // BLK, 8, BLK), jnp.float32),
        compiler_params=pltpu.CompilerParams(
            use_tc_tiling_on_sc=False, needs_layout_passes=False),
        scratch_types=[
            pltpu.VMEM((BLK,), jnp.int32),
            pltpu.VMEM((BLK,), jnp.int32),
            pltpu.VMEM((BLK,), jnp.int32),
            pltpu.VMEM((BLK, 2 * DIM), jnp.float32),
            pltpu.VMEM((8, 8, BLK), jnp.float32),
            pltpu.SemaphoreType.DMA,
        ],
    )
    def gather_kernel(xq_hbm, tp_hbm, out_hbm, idxb, pidx, selv, rows_v,
                      trans_v, sem):
        wid = lax.axis_index("s") * NC + lax.axis_index("c")
        iota = lax.iota(jnp.int32, L)

        def block(t, carry):
            bid = wid * per_w + t
            base = pl.multiple_of(bid * BLK, BLK)
            pltpu.sync_copy(xq_hbm.at[pl.ds(base, BLK)], idxb)
            for cg in range(BLK // L):
                v = idxb[pl.ds(cg * L, L)]
                pidx[pl.ds(cg * L, L)] = lax.shift_right_logical(v, 1)
                selv[pl.ds(cg * L, L)] = lax.shift_left(
                    lax.bitwise_and(v, 1), 6)
            pltpu.async_copy(tp_hbm.at[pidx], rows_v, sem).wait()
            for cg in range(BLK // L):
                rows16 = iota + cg * L
                sel16 = selv[pl.ds(cg * L, L)]
                for d in range(DIM):
                    vals = plsc.load_gather(rows_v, [rows16, sel16 + d])
                    trans_v[d // 8, d % 8, pl.ds(cg * L, L)] = vals
            h = bid // (B // BLK)
            bc = lax.rem(bid, B // BLK)
            pltpu.sync_copy(trans_v, out_hbm.at[h, :, bc])
            return carry

        lax.fori_loop(0, per_w, block, 0)

    return gather_kernel


def kernel(x, table):
    B, H = x.shape
    vocab = table.shape[0]
    xq = x.T.reshape(B * H).astype(jnp.int32)
    tp = table.reshape(vocab // 2, 2 * DIM)
    out5 = _build_gather(B, H, vocab)(xq, tp)
    return out5.transpose(2, 4, 0, 1, 3).reshape(B, H, DIM)


# speedup vs baseline: 1.1709x; 1.1709x over previous
import functools

import jax
import jax.numpy as jnp
from jax import lax
from jax.experimental import pallas as pl
from jax.experimental.pallas import tpu as pltpu
from jax.experimental.pallas import tpu_sc as plsc

DIM = 64
NC, NS, L = 2, 16, 16
NW = NC * NS
BLK = 128
NBUF = 2


@functools.lru_cache(maxsize=None)
def _build_gather(B, H, vocab):
    nbc = B // BLK
    n_blocks = nbc * H
    per_w = n_blocks // NW
    n_outer = per_w // NBUF
    mesh = plsc.VectorSubcoreMesh(core_axis_name="c", subcore_axis_name="s")

    @functools.partial(
        pl.kernel,
        mesh=mesh,
        out_type=jax.ShapeDtypeStruct((H, 8, nbc, 8, BLK), jnp.float32),
        compiler_params=pltpu.CompilerParams(
            use_tc_tiling_on_sc=False, needs_layout_passes=False),
        scratch_types=[
            pltpu.VMEM((per_w * BLK,), jnp.int32),
            pltpu.VMEM((NBUF, BLK), jnp.int32),
            pltpu.VMEM((NBUF, BLK), jnp.int32),
            pltpu.VMEM((NBUF, BLK, 2 * DIM), jnp.float32),
            pltpu.VMEM((NBUF, 8, 8, BLK), jnp.float32),
            pltpu.SemaphoreType.DMA((NBUF,)),
            pltpu.SemaphoreType.DMA((NBUF,)),
        ],
    )
    def gather_kernel(xq_hbm, tp_hbm, out_hbm, idx_all, pidx, selv, rows_v,
                      trans_v, gsem, osem):
        wid = lax.axis_index("s") * NC + lax.axis_index("c")
        iota = lax.iota(jnp.int32, L)
        pltpu.sync_copy(
            xq_hbm.at[pl.ds(pl.multiple_of(wid * per_w * BLK, BLK),
                            per_w * BLK)],
            idx_all)

        def stage_in(t, b):
            off = pl.multiple_of(t * BLK, BLK)
            for cg in range(BLK // L):
                v = idx_all[pl.ds(off + cg * L, L)]
                pidx[b, pl.ds(cg * L, L)] = lax.shift_right_logical(v, 1)
                selv[b, pl.ds(cg * L, L)] = lax.shift_left(
                    lax.bitwise_and(v, 1), 6)
            pltpu.async_copy(tp_hbm.at[pidx.at[b]], rows_v.at[b], gsem.at[b])

        def out_descr(t, b):
            bid = wid * per_w + t
            h = bid // nbc
            bc = lax.rem(bid, nbc)
            return pltpu.make_async_copy(
                trans_v.at[b], out_hbm.at[h, :, bc], osem.at[b])

        def transpose_and_out(t, b):
            for cg in range(BLK // L):
                rows16 = iota + cg * L
                sel16 = selv[b, pl.ds(cg * L, L)]
                for d in range(DIM):
                    vals = plsc.load_gather(
                        rows_v.at[b], [rows16, sel16 + d])
                    trans_v[b, d // 8, d % 8, pl.ds(cg * L, L)] = vals
            out_descr(t, b).start()

        stage_in(0, 0)

        def outer(o, carry):
            for b in range(NBUF):
                t = o * NBUF + b
                nb = (b + 1) % NBUF
                if b == NBUF - 1:
                    @pl.when(o < n_outer - 1)
                    def _():
                        stage_in(t + 1, nb)
                else:
                    stage_in(t + 1, nb)
                pltpu.make_async_copy(
                    tp_hbm.at[pidx.at[b]], rows_v.at[b], gsem.at[b]).wait()

                @pl.when(o > 0)
                def _():
                    out_descr(t, b).wait()
                transpose_and_out(t, b)
            return carry

        lax.fori_loop(0, n_outer, outer, 0)
        for b in range(NBUF):
            out_descr(per_w - NBUF + b, b).wait()

    return gather_kernel


def kernel(x, table):
    B, H = x.shape
    vocab = table.shape[0]
    xq = x.T.reshape(B * H).astype(jnp.int32)
    tp = table.reshape(vocab // 2, 2 * DIM)
    out5 = _build_gather(B, H, vocab)(xq, tp)
    return out5.transpose(2, 4, 0, 1, 3).reshape(B, H, DIM)


# speedup vs baseline: 1.8013x; 1.5384x over previous
import functools

import jax
import jax.numpy as jnp
from jax import lax
from jax.experimental import pallas as pl
from jax.experimental.pallas import tpu as pltpu
from jax.experimental.pallas import tpu_sc as plsc

DIM = 64
NC, NS, L = 2, 16, 16
NW = NC * NS
BLK = 128
NBUF = 2


@functools.lru_cache(maxsize=None)
def _build_gather(B, H, vocab):
    nbc = B // BLK
    n_blocks = nbc * H
    per_w = n_blocks // NW
    n_outer = per_w // NBUF
    mesh = plsc.VectorSubcoreMesh(core_axis_name="c", subcore_axis_name="s")

    @functools.partial(
        pl.kernel,
        mesh=mesh,
        out_type=jax.ShapeDtypeStruct((H, 8, nbc, 8, BLK), jnp.float32),
        compiler_params=pltpu.CompilerParams(
            use_tc_tiling_on_sc=False, needs_layout_passes=False),
        scratch_types=[
            pltpu.VMEM((per_w * BLK,), jnp.int32),
            pltpu.VMEM((NBUF, BLK), jnp.int32),
            pltpu.VMEM((NBUF, BLK), jnp.int32),
            pltpu.VMEM((NBUF, BLK, 2 * DIM), jnp.float32),
            pltpu.VMEM((NBUF, 8, 8, BLK), jnp.float32),
            pltpu.SemaphoreType.DMA((NBUF,)),
            pltpu.SemaphoreType.DMA((NBUF,)),
        ],
    )
    def gather_kernel(xq_hbm, tp_hbm, out_hbm, idx_all, pidx, selv, rows_v,
                      trans_v, gsem, osem):
        wid = lax.axis_index("s") * NC + lax.axis_index("c")
        iota = lax.iota(jnp.int32, L)
        pltpu.sync_copy(
            xq_hbm.at[pl.ds(pl.multiple_of(wid * per_w * BLK, BLK),
                            per_w * BLK)],
            idx_all)

        def stage_in(t, b):
            off = pl.multiple_of(t * BLK, BLK)
            for cg in range(BLK // L):
                v = idx_all[pl.ds(off + cg * L, L)]
                pidx[b, pl.ds(cg * L, L)] = lax.shift_right_logical(v, 1)
                selv[b, pl.ds(cg * L, L)] = lax.shift_left(
                    lax.bitwise_and(v, 1), 6)
            pltpu.async_copy(tp_hbm.at[pidx.at[b]], rows_v.at[b], gsem.at[b])

        def out_descr(t, b):
            bid = wid * per_w + t
            h = bid // nbc
            bc = lax.rem(bid, nbc)
            return pltpu.make_async_copy(
                trans_v.at[b], out_hbm.at[h, :, bc], osem.at[b])

        def transpose_and_out(t, b):
            rows16s = [iota + cg * L for cg in range(BLK // L)]
            sel16s = [selv[b, pl.ds(cg * L, L)] for cg in range(BLK // L)]

            @plsc.parallel_loop(0, DIM)
            def _(d):
                br = lax.div(d, 8)
                r = lax.rem(d, 8)
                for cg in range(BLK // L):
                    vals = plsc.load_gather(
                        rows_v.at[b], [rows16s[cg], sel16s[cg] + d])
                    trans_v[b, br, r, pl.ds(cg * L, L)] = vals

            out_descr(t, b).start()

        stage_in(0, 0)

        def outer(o, carry):
            for b in range(NBUF):
                t = o * NBUF + b
                nb = (b + 1) % NBUF
                if b == NBUF - 1:
                    @pl.when(o < n_outer - 1)
                    def _():
                        stage_in(t + 1, nb)
                else:
                    stage_in(t + 1, nb)
                pltpu.make_async_copy(
                    tp_hbm.at[pidx.at[b]], rows_v.at[b], gsem.at[b]).wait()

                @pl.when(o > 0)
                def _():
                    out_descr(t, b).wait()
                transpose_and_out(t, b)
            return carry

        lax.fori_loop(0, n_outer, outer, 0)
        for b in range(NBUF):
            out_descr(per_w - NBUF + b, b).wait()

    return gather_kernel


def kernel(x, table):
    B, H = x.shape
    vocab = table.shape[0]
    xq = x.T.reshape(B * H).astype(jnp.int32)
    tp = table.reshape(vocab // 2, 2 * DIM)
    out5 = _build_gather(B, H, vocab)(xq, tp)
    return out5.transpose(2, 4, 0, 1, 3).reshape(B, H, DIM)


# speedup vs baseline: 1.8136x; 1.0068x over previous
import functools

import jax
import jax.numpy as jnp
from jax import lax
from jax.experimental import pallas as pl
from jax.experimental.pallas import tpu as pltpu
from jax.experimental.pallas import tpu_sc as plsc

DIM = 64
NC, NS, L = 2, 16, 16
NW = NC * NS
BLK = 128
NBUF = 2


@functools.lru_cache(maxsize=None)
def _build_gather(B, H, vocab):
    nbc = B // BLK
    n_blocks = nbc * H
    per_w = n_blocks // NW
    n_outer = per_w // NBUF
    mesh = plsc.VectorSubcoreMesh(core_axis_name="c", subcore_axis_name="s")

    @functools.partial(
        pl.kernel,
        mesh=mesh,
        out_type=jax.ShapeDtypeStruct((H, 8, nbc, 8, BLK), jnp.float32),
        compiler_params=pltpu.CompilerParams(
            use_tc_tiling_on_sc=False, needs_layout_passes=False),
        scratch_types=[
            pltpu.VMEM((per_w * BLK,), jnp.int32),
            pltpu.VMEM((NBUF, BLK, DIM), jnp.float32),
            pltpu.VMEM((NBUF, 8, 8, BLK), jnp.float32),
            pltpu.SemaphoreType.DMA((NBUF,)),
            pltpu.SemaphoreType.DMA((NBUF,)),
        ],
    )
    def gather_kernel(xq_hbm, tp_hbm, out_hbm, idx_all, rows_v,
                      trans_v, gsem, osem):
        wid = lax.axis_index("s") * NC + lax.axis_index("c")
        iota = lax.iota(jnp.int32, L)
        zero = iota * 0
        pltpu.sync_copy(
            xq_hbm.at[pl.ds(pl.multiple_of(wid * per_w * BLK, BLK),
                            per_w * BLK)],
            idx_all)

        def gather_descr(t, b):
            off = pl.multiple_of(t * BLK, BLK)
            return pltpu.make_async_copy(
                tp_hbm.at[idx_all.at[pl.ds(off, BLK)]], rows_v.at[b],
                gsem.at[b])

        def out_descr(t, b):
            bid = wid * per_w + t
            h = bid // nbc
            bc = lax.rem(bid, nbc)
            return pltpu.make_async_copy(
                trans_v.at[b], out_hbm.at[h, :, bc], osem.at[b])

        def transpose_and_out(t, b):
            rows16s = [iota + cg * L for cg in range(BLK // L)]

            @plsc.parallel_loop(0, DIM)
            def _(d):
                br = lax.div(d, 8)
                r = lax.rem(d, 8)
                dv = zero + d
                for cg in range(BLK // L):
                    vals = plsc.load_gather(
                        rows_v.at[b], [rows16s[cg], dv])
                    trans_v[b, br, r, pl.ds(cg * L, L)] = vals

            out_descr(t, b).start()

        gather_descr(0, 0).start()

        def outer(o, carry):
            for b in range(NBUF):
                t = o * NBUF + b
                nb = (b + 1) % NBUF
                if b == NBUF - 1:
                    @pl.when(o < n_outer - 1)
                    def _():
                        gather_descr(t + 1, nb).start()
                else:
                    gather_descr(t + 1, nb).start()
                gather_descr(t, b).wait()

                @pl.when(o > 0)
                def _():
                    out_descr(t, b).wait()
                transpose_and_out(t, b)
            return carry

        lax.fori_loop(0, n_outer, outer, 0)
        for b in range(NBUF):
            out_descr(per_w - NBUF + b, b).wait()

    return gather_kernel


def kernel(x, table):
    B, H = x.shape
    vocab = table.shape[0]
    xq = x.T.reshape(B * H).astype(jnp.int32)
    out5 = _build_gather(B, H, vocab)(xq, table)
    return out5.transpose(2, 4, 0, 1, 3).reshape(B, H, DIM)


# speedup vs baseline: 1.9379x; 1.0685x over previous
import functools

import jax
import jax.numpy as jnp
from jax import lax
from jax.experimental import pallas as pl
from jax.experimental.pallas import tpu as pltpu
from jax.experimental.pallas import tpu_sc as plsc

DIM = 64
NC, NS, L = 2, 16, 16
NW = NC * NS
BLK = 128
NBUF = 2


@functools.lru_cache(maxsize=None)
def _build_gather(B, H, vocab):
    nbc = B // BLK
    n_blocks = nbc * H
    per_w = n_blocks // NW
    n_outer = per_w // NBUF
    mesh = plsc.VectorSubcoreMesh(core_axis_name="c", subcore_axis_name="s")

    @functools.partial(
        pl.kernel,
        mesh=mesh,
        out_type=jax.ShapeDtypeStruct((H, 8, nbc, 8, BLK), jnp.float32),
        compiler_params=pltpu.CompilerParams(
            use_tc_tiling_on_sc=False, needs_layout_passes=False),
        scratch_types=[
            pltpu.VMEM((per_w * BLK,), jnp.int32),
            pltpu.VMEM((NBUF, BLK, DIM), jnp.float32),
            pltpu.VMEM((NBUF, 8, 8, BLK), jnp.float32),
            pltpu.SemaphoreType.DMA((NBUF,)),
            pltpu.SemaphoreType.DMA((NBUF,)),
        ],
    )
    def gather_kernel(xq_hbm, tp_hbm, out_hbm, idx_all, rows_v,
                      trans_v, gsem, osem):
        wid = lax.axis_index("s") * NC + lax.axis_index("c")
        iota = lax.iota(jnp.int32, L)
        zero = iota * 0
        pltpu.sync_copy(
            xq_hbm.at[pl.ds(pl.multiple_of(wid * per_w * BLK, BLK),
                            per_w * BLK)],
            idx_all)

        def gather_descr(t, b):
            off = pl.multiple_of(t * BLK, BLK)
            return pltpu.make_async_copy(
                tp_hbm.at[idx_all.at[pl.ds(off, BLK)]], rows_v.at[b],
                gsem.at[b])

        def out_descr(t, b):
            bid = wid * per_w + t
            h = bid // nbc
            bc = lax.rem(bid, nbc)
            return pltpu.make_async_copy(
                trans_v.at[b], out_hbm.at[h, :, bc], osem.at[b])

        def transpose_and_out(t, b):
            rows16s = [iota + cg * L for cg in range(BLK // L)]

            @plsc.parallel_loop(0, DIM, unroll=8)
            def _(d):
                br = lax.div(d, 8)
                r = lax.rem(d, 8)
                dv = zero + d
                for cg in range(BLK // L):
                    vals = plsc.load_gather(
                        rows_v.at[b], [rows16s[cg], dv])
                    trans_v[b, br, r, pl.ds(cg * L, L)] = vals

            out_descr(t, b).start()

        gather_descr(0, 0).start()

        def outer(o, carry):
            for b in range(NBUF):
                t = o * NBUF + b
                nb = (b + 1) % NBUF
                if b == NBUF - 1:
                    @pl.when(o < n_outer - 1)
                    def _():
                        gather_descr(t + 1, nb).start()
                else:
                    gather_descr(t + 1, nb).start()
                gather_descr(t, b).wait()

                @pl.when(o > 0)
                def _():
                    out_descr(t, b).wait()
                transpose_and_out(t, b)
            return carry

        lax.fori_loop(0, n_outer, outer, 0)
        for b in range(NBUF):
            out_descr(per_w - NBUF + b, b).wait()

    return gather_kernel


def kernel(x, table):
    B, H = x.shape
    vocab = table.shape[0]
    xq = x.T.reshape(B * H).astype(jnp.int32)
    out5 = _build_gather(B, H, vocab)(xq, table)
    return out5.transpose(2, 4, 0, 1, 3).reshape(B, H, DIM)
